# baseline (speedup 1.0000x reference)
import functools

import jax
import jax.numpy as jnp
from jax import lax
from jax.experimental import pallas as pl
from jax.experimental.pallas import tpu as pltpu
from jax.experimental.pallas import tpu_sc as plsc

N_USERS = 1000000
N_ITEMS = 1000000
DIM = 64
BATCH = 16384
MAX_NORM = 2.0

NC = 2
NS = 16
L = 16
NW = NC * NS
B_PER_W = BATCH // NW


def _rsqrt_newton(x):
    i = lax.bitcast_convert_type(x, jnp.int32)
    i = jnp.int32(0x5F3759DF) - lax.shift_right_arithmetic(i, 1)
    y = lax.bitcast_convert_type(i, jnp.float32)
    xh = x * 0.5
    for _ in range(3):
        y = y * (1.5 - xh * y * y)
    return y


def _sc_body(uid_hbm, iid_hbm, utab_hbm, itab_hbm, out_hbm,
             uidx_v, iidx_v, urows_v, irows_v, out_v, sem_u, sem_i):
    wid = lax.axis_index("s") * NC + lax.axis_index("c")
    base = wid * B_PER_W

    pltpu.sync_copy(uid_hbm.at[pl.ds(base, B_PER_W)], uidx_v)
    pltpu.sync_copy(iid_hbm.at[pl.ds(base, B_PER_W)], iidx_v)
    cu = pltpu.async_copy(utab_hbm.at[uidx_v], urows_v, sem_u)
    ci = pltpu.async_copy(itab_hbm.at[iidx_v], irows_v, sem_i)
    cu.wait()
    ci.wait()

    ids16 = lax.iota(jnp.int32, L)
    zeros = jnp.zeros((L,), jnp.float32)

    def group_body(g, _):
        rows = g * L + ids16

        def feat_body(j, carry):
            uu, vv, uv = carry
            cols = jnp.full((L,), 0, jnp.int32) + j
            u = plsc.load_gather(urows_v, [rows, cols])
            v = plsc.load_gather(irows_v, [rows, cols])
            return (uu + u * u, vv + v * v, uv + u * v)

        uu, vv, uv = lax.fori_loop(0, DIM, feat_body, (zeros, zeros, zeros),
                                   unroll=True)

        su = jnp.where(uu > MAX_NORM * MAX_NORM,
                       MAX_NORM * _rsqrt_newton(uu), 1.0)
        sv = jnp.where(vv > MAX_NORM * MAX_NORM,
                       MAX_NORM * _rsqrt_newton(vv), 1.0)
        dot = su * sv * uv
        rating = 5.0 / (1.0 + jnp.exp(-dot))
        plsc.store_scatter(out_v, [rows], rating)
        return 0

    lax.fori_loop(0, B_PER_W // L, group_body, 0)

    pltpu.sync_copy(out_v, out_hbm.at[pl.ds(base, B_PER_W)])


@jax.jit
def kernel(user_id, item_id, users_table, items_table):
    mesh = plsc.VectorSubcoreMesh(core_axis_name="c", subcore_axis_name="s")
    fn = functools.partial(
        pl.kernel,
        out_type=jax.ShapeDtypeStruct((BATCH,), jnp.float32),
        mesh=mesh,
        compiler_params=pltpu.CompilerParams(needs_layout_passes=False,
                                             use_tc_tiling_on_sc=False),
        scratch_types=[
            pltpu.VMEM((B_PER_W,), jnp.int32),
            pltpu.VMEM((B_PER_W,), jnp.int32),
            pltpu.VMEM((B_PER_W, DIM), jnp.float32),
            pltpu.VMEM((B_PER_W, DIM), jnp.float32),
            pltpu.VMEM((B_PER_W,), jnp.float32),
            pltpu.SemaphoreType.DMA,
            pltpu.SemaphoreType.DMA,
        ],
    )(_sc_body)
    return fn(user_id, item_id, users_table, items_table)
